# Initial kernel scaffold; baseline (speedup 1.0000x reference)
import jax
import jax.numpy as jnp
from jax.experimental import pallas as pl


def kernel(discharge, overburden, melt_rate, status_at_link, node_at_link_head, node_at_link_tail, links_at_node):
    raise NotImplementedError("write your pallas kernel here")


# speedup vs baseline: 75.3772x; 75.3772x over previous
import functools

import jax
import jax.numpy as jnp
from jax import lax
from jax.experimental import pallas as pl
from jax.experimental.pallas import tpu as pltpu
from jax.experimental.pallas import tpu_sc as plsc

N = 100000
E = 200000
NC = 2
NS = 16
NW = NC * NS

LINK_CHUNK = 6272
NODE_CHUNK = 3136

@functools.cache
def _mesh():
    return plsc.VectorSubcoreMesh(core_axis_name="c", subcore_axis_name="s",
                                  num_cores=NC, num_subcores=NS)


def _wid():
    return lax.axis_index("s") * NC + lax.axis_index("c")


def _signed_body(disch, status, head, tail, over, signed_out,
                 d_v, s_v, h_v, t_v, oh_v, ot_v, o_v, sem):
    base = jnp.minimum(_wid() * LINK_CHUNK, E - LINK_CHUNK)
    base = pl.multiple_of(base, 8)
    sl = pl.ds(base, LINK_CHUNK)
    pltpu.sync_copy(disch.at[sl], d_v)
    pltpu.sync_copy(status.at[sl], s_v)
    pltpu.sync_copy(head.at[sl], h_v)
    pltpu.sync_copy(tail.at[sl], t_v)
    cp1 = pltpu.async_copy(over.at[h_v], oh_v, sem)
    cp2 = pltpu.async_copy(over.at[t_v], ot_v, sem)
    cp1.wait()
    cp2.wait()

    def body(i, carry):
        v = pl.ds(i * 16, 16)
        d = jnp.where(s_v[v] == 4, 0.0, d_v[v])
        sign = jnp.where(oh_v[v] > ot_v[v], 1.0, -1.0)
        o_v[v] = sign * d
        return carry

    lax.fori_loop(0, LINK_CHUNK // 16, body, 0)
    pltpu.sync_copy(o_v, signed_out.at[sl])


def _flux_body(links_t, signed, melt, out, idx_v, g_v, m_v, o_v, sem):
    base = jnp.minimum(_wid() * NODE_CHUNK, N - NODE_CHUNK)
    base = pl.multiple_of(base, 8)
    for l in range(4):
        pltpu.sync_copy(links_t.at[pl.ds(l * N + base, NODE_CHUNK)],
                        idx_v.at[pl.ds(l * NODE_CHUNK, NODE_CHUNK)])
    pltpu.async_copy(signed.at[idx_v], g_v, sem).wait()
    pltpu.sync_copy(melt.at[pl.ds(base, NODE_CHUNK)], m_v)

    def body(j, carry):
        nb = j * 16
        acc = (g_v[pl.ds(nb, 16)]
               + g_v[pl.ds(NODE_CHUNK + nb, 16)]
               + g_v[pl.ds(2 * NODE_CHUNK + nb, 16)]
               + g_v[pl.ds(3 * NODE_CHUNK + nb, 16)])
        o_v[pl.ds(nb, 16)] = acc - m_v[pl.ds(nb, 16)]
        return carry

    lax.fori_loop(0, NODE_CHUNK // 16, body, 0)
    pltpu.sync_copy(o_v, out.at[pl.ds(base, NODE_CHUNK)])


@functools.cache
def _signed_call():
    return pl.kernel(
        _signed_body,
        out_type=jax.ShapeDtypeStruct((E,), jnp.float32),
        mesh=_mesh(),
        scratch_types=[
            pltpu.VMEM((LINK_CHUNK,), jnp.float32),
            pltpu.VMEM((LINK_CHUNK,), jnp.int32),
            pltpu.VMEM((LINK_CHUNK,), jnp.int32),
            pltpu.VMEM((LINK_CHUNK,), jnp.int32),
            pltpu.VMEM((LINK_CHUNK,), jnp.float32),
            pltpu.VMEM((LINK_CHUNK,), jnp.float32),
            pltpu.VMEM((LINK_CHUNK,), jnp.float32),
            pltpu.SemaphoreType.DMA,
        ],
    )


@functools.cache
def _flux_call():
    return pl.kernel(
        _flux_body,
        out_type=jax.ShapeDtypeStruct((N,), jnp.float32),
        mesh=_mesh(),
        scratch_types=[
            pltpu.VMEM((NODE_CHUNK * 4,), jnp.int32),
            pltpu.VMEM((NODE_CHUNK * 4,), jnp.float32),
            pltpu.VMEM((NODE_CHUNK,), jnp.float32),
            pltpu.VMEM((NODE_CHUNK,), jnp.float32),
            pltpu.SemaphoreType.DMA,
        ],
    )


def kernel(discharge, overburden, melt_rate, status_at_link,
           node_at_link_head, node_at_link_tail, links_at_node):
    status = status_at_link.astype(jnp.int32)
    head = node_at_link_head.astype(jnp.int32)
    tail = node_at_link_tail.astype(jnp.int32)
    links_t = links_at_node.astype(jnp.int32).T.reshape(4 * N)
    signed = _signed_call()(discharge, status, head, tail, overburden)
    return _flux_call()(links_t, signed, melt_rate)
